# baseline (speedup 1.0000x reference)
import jax
import jax.numpy as jnp
from jax.experimental import pallas as pl

ROWS = 16384
COLS = 200
DIM = 128
FLAT = ROWS * COLS
BLOCK_F = 25600
NBLOCKS = FLAT // BLOCK_F


def _mask_embed_kernel(mask_ref, emb_ref, out_ref):
    mb = mask_ref[0].astype(jnp.bfloat16)
    lhs = jnp.broadcast_to(mb, (3, BLOCK_F))
    e0 = emb_ref[0, :]
    d = emb_ref[1, :] - e0
    d_hi = d.astype(jnp.bfloat16)
    r1 = d - d_hi.astype(jnp.float32)
    d_mid = r1.astype(jnp.bfloat16)
    d_lo = (r1 - d_mid.astype(jnp.float32)).astype(jnp.bfloat16)
    rhs = jnp.stack([d_hi, d_mid, d_lo], axis=0)
    outer = jax.lax.dot_general(
        lhs, rhs,
        dimension_numbers=(((0,), (0,)), ((), ())),
        preferred_element_type=jnp.float32,
    )
    out_ref[...] = outer + e0[None, :]


def kernel(mask01, emb):
    mask_flat = mask01.reshape(NBLOCKS, 1, BLOCK_F)
    out = pl.pallas_call(
        _mask_embed_kernel,
        grid=(NBLOCKS,),
        in_specs=[
            pl.BlockSpec((1, 1, BLOCK_F), lambda i: (i, 0, 0)),
            pl.BlockSpec((2, DIM), lambda i: (0, 0)),
        ],
        out_specs=pl.BlockSpec((BLOCK_F, DIM), lambda i: (i, 0)),
        out_shape=jax.ShapeDtypeStruct((FLAT, DIM), jnp.float32),
    )(mask_flat, emb)
    return out.reshape(ROWS, COLS, DIM)


# speedup vs baseline: 1.1124x; 1.1124x over previous
import jax
import jax.numpy as jnp
from jax.experimental import pallas as pl

ROWS = 16384
COLS = 200
DIM = 128
BLOCK_ROWS = 128


def _mask_embed_kernel(mask_ref, emb_ref, out_ref):
    m = mask_ref[...]
    e0 = emb_ref[0, :]
    e1 = emb_ref[1, :]
    out_ref[...] = jnp.where((m[:, :, None] != 0), e1[None, None, :],
                             e0[None, None, :])


def kernel(mask01, emb):
    grid = (ROWS // BLOCK_ROWS,)
    return pl.pallas_call(
        _mask_embed_kernel,
        grid=grid,
        in_specs=[
            pl.BlockSpec((BLOCK_ROWS, COLS), lambda i: (i, 0)),
            pl.BlockSpec((2, DIM), lambda i: (0, 0)),
        ],
        out_specs=pl.BlockSpec((BLOCK_ROWS, COLS, DIM), lambda i: (i, 0, 0)),
        out_shape=jax.ShapeDtypeStruct((ROWS, COLS, DIM), jnp.float32),
    )(mask01, emb)
